# baseline (speedup 1.0000x reference)
import functools

import jax
import jax.numpy as jnp
from jax.experimental import pallas as pl
from jax.experimental.pallas import tpu as pltpu


def _row_tile(n: int, target: int = 512) -> int:
    best = 8 if n % 8 == 0 else 1
    for t in range(8, target + 1, 8):
        if n % t == 0:
            best = t
    return best


def _gcn_body(seq_ref, adj_ref, w_ref, bias_ref, a_ref, out_ref, fts_ref):
    @pl.when(pl.program_id(0) == 0)
    def _():
        fts_ref[...] = jnp.dot(
            seq_ref[...], w_ref[...].T, preferred_element_type=jnp.float32
        )

    out = jnp.dot(adj_ref[...], fts_ref[...], preferred_element_type=jnp.float32)
    out = out + bias_ref[...]
    a = a_ref[0, 0]
    out_ref[...] = jnp.maximum(out, 0.0) + a * jnp.minimum(out, 0.0)


@functools.partial(jax.jit, static_argnames=())
def kernel(seq, adj, W, bias, prelu_a):
    b, n, in_ft = seq.shape
    out_ft = W.shape[0]
    seq2 = seq.reshape(b * n, in_ft)
    adj2 = adj.reshape(b * n, n)
    bias2 = bias.reshape(1, out_ft)
    a2 = jnp.asarray(prelu_a, jnp.float32).reshape(1, 1)

    tile_m = 512
    grid = (pl.cdiv(n, tile_m),)

    out = pl.pallas_call(
        _gcn_body,
        grid=grid,
        in_specs=[
            pl.BlockSpec((n, in_ft), lambda i: (0, 0)),
            pl.BlockSpec((tile_m, n), lambda i: (i, 0)),
            pl.BlockSpec((out_ft, in_ft), lambda i: (0, 0)),
            pl.BlockSpec((1, out_ft), lambda i: (0, 0)),
            pl.BlockSpec((1, 1), lambda i: (0, 0)),
        ],
        out_specs=pl.BlockSpec((tile_m, out_ft), lambda i: (i, 0)),
        out_shape=jax.ShapeDtypeStruct((n, out_ft), jnp.float32),
        scratch_shapes=[pltpu.VMEM((n, out_ft), jnp.float32)],
        compiler_params=pltpu.CompilerParams(
            dimension_semantics=("arbitrary",),
            vmem_limit_bytes=64 * 1024 * 1024,
        ),
    )(seq2, adj2, W, bias2, a2)
    return out.reshape(b, n, out_ft)


# speedup vs baseline: 1.0084x; 1.0084x over previous
import functools

import jax
import jax.numpy as jnp
from jax.experimental import pallas as pl
from jax.experimental.pallas import tpu as pltpu


def _row_tile(n: int, target: int = 512) -> int:
    best = 8 if n % 8 == 0 else 1
    for t in range(8, target + 1, 8):
        if n % t == 0:
            best = t
    return best


def _gcn_body(seq_ref, adja_ref, adjb_ref, w_ref, bias_ref, a_ref,
              out_ref, fts_ref):
    @pl.when(pl.program_id(0) == 0)
    def _():
        fts_ref[...] = jnp.dot(
            seq_ref[...], w_ref[...].T, preferred_element_type=jnp.float32
        )

    a = a_ref[0, 0]
    bias = bias_ref[...]
    fts = fts_ref[...]
    oa = jnp.dot(adja_ref[...], fts, preferred_element_type=jnp.float32) + bias
    out_ref[0] = jnp.maximum(oa, 0.0) + a * jnp.minimum(oa, 0.0)
    ob = jnp.dot(adjb_ref[...], fts, preferred_element_type=jnp.float32) + bias
    out_ref[1] = jnp.maximum(ob, 0.0) + a * jnp.minimum(ob, 0.0)


@functools.partial(jax.jit, static_argnames=())
def kernel(seq, adj, W, bias, prelu_a):
    b, n, in_ft = seq.shape
    out_ft = W.shape[0]
    seq2 = seq.reshape(b * n, in_ft)
    adj2 = adj.reshape(b * n, n)
    bias2 = bias.reshape(1, out_ft)
    a2 = jnp.asarray(prelu_a, jnp.float32).reshape(1, 1)

    half = n // 2
    tile_m = _row_tile(half, target=200)
    steps = half // tile_m
    off = steps

    out = pl.pallas_call(
        _gcn_body,
        grid=(steps,),
        in_specs=[
            pl.BlockSpec((n, in_ft), lambda i: (0, 0)),
            pl.BlockSpec((tile_m, n), lambda i: (i, 0)),
            pl.BlockSpec((tile_m, n), lambda i: (i + off, 0)),
            pl.BlockSpec((out_ft, in_ft), lambda i: (0, 0)),
            pl.BlockSpec((1, out_ft), lambda i: (0, 0)),
            pl.BlockSpec((1, 1), lambda i: (0, 0)),
        ],
        out_specs=pl.BlockSpec((2, tile_m, out_ft), lambda i: (0, i, 0)),
        out_shape=jax.ShapeDtypeStruct((2, half, out_ft), jnp.float32),
        scratch_shapes=[pltpu.VMEM((n, out_ft), jnp.float32)],
        compiler_params=pltpu.CompilerParams(
            dimension_semantics=("arbitrary",),
            vmem_limit_bytes=64 * 1024 * 1024,
        ),
    )(seq2, adj2, adj2, W, bias2, a2)
    return out.reshape(b, n, out_ft)


# speedup vs baseline: 1.0228x; 1.0143x over previous
import functools

import jax
import jax.numpy as jnp
from jax.experimental import pallas as pl
from jax.experimental.pallas import tpu as pltpu


def _row_tile(n: int, target: int = 512) -> int:
    best = 8 if n % 8 == 0 else 1
    for t in range(8, target + 1, 8):
        if n % t == 0:
            best = t
    return best


def _gcn_body(seq_ref, adj_ref, w_ref, bias_ref, a_ref, out_ref, fts_ref):
    @pl.when(pl.program_id(0) == 0)
    def _():
        fts_ref[...] = jnp.dot(
            seq_ref[...], w_ref[...].T, preferred_element_type=jnp.float32
        )

    out = jnp.dot(adj_ref[...], fts_ref[...], preferred_element_type=jnp.float32)
    out = out + bias_ref[...]
    a = a_ref[0, 0]
    out_ref[...] = jnp.maximum(out, 0.0) + a * jnp.minimum(out, 0.0)


@functools.partial(jax.jit, static_argnames=())
def kernel(seq, adj, W, bias, prelu_a):
    b, n, in_ft = seq.shape
    out_ft = W.shape[0]
    seq2 = seq.reshape(b * n, in_ft)
    adj2 = adj.reshape(b * n, n)
    bias2 = bias.reshape(1, out_ft)
    a2 = jnp.asarray(prelu_a, jnp.float32).reshape(1, 1)

    tile_m = _row_tile(n, target=400)
    grid = (n // tile_m,)

    out = pl.pallas_call(
        _gcn_body,
        grid=grid,
        in_specs=[
            pl.BlockSpec((n, in_ft), lambda i: (0, 0)),
            pl.BlockSpec((tile_m, n), lambda i: (i, 0)),
            pl.BlockSpec((out_ft, in_ft), lambda i: (0, 0)),
            pl.BlockSpec((1, out_ft), lambda i: (0, 0)),
            pl.BlockSpec((1, 1), lambda i: (0, 0)),
        ],
        out_specs=pl.BlockSpec((tile_m, out_ft), lambda i: (i, 0)),
        out_shape=jax.ShapeDtypeStruct((n, out_ft), jnp.float32),
        scratch_shapes=[pltpu.VMEM((n, out_ft), jnp.float32)],
        compiler_params=pltpu.CompilerParams(
            dimension_semantics=("arbitrary",),
            vmem_limit_bytes=64 * 1024 * 1024,
        ),
    )(seq2, adj2, W, bias2, a2)
    return out.reshape(b, n, out_ft)
